# baseline (speedup 1.0000x reference)
import jax
import jax.numpy as jnp
from jax import lax
from jax.experimental import pallas as pl
from jax.experimental.pallas import tpu as pltpu
from jax.experimental.pallas import tpu_sc as plsc

B = 4096
SEQ = 200
D = 32
OUT_DIM = 128
V = 1000000

NC = 2
NS = 16
NW = NC * NS
RPW = B // NW
IPW = RPW * SEQ
UNIT = 128
UPC = 8
CHUNK_ROWS = UPC * UNIT
CHUNKS = IPW // CHUNK_ROWS

NBLK = V // UNIT
TAIL = V - NBLK * UNIT
GB = 8
NBUF = 2
BPW = 244
NBATCH = 30
LPW = BPW - NBATCH * GB
REM = NBLK - NW * BPW


def _tr_compute(in_ref, out_ref, iota16):
    @pl.loop(0, 32, unroll=4)
    def _row(k):
        for b in range(GB):
            for g in range(8):
                vals = plsc.load_gather(
                    in_ref,
                    [16 * (g % 2) + iota16,
                     jnp.broadcast_to(b * UNIT + 4 * k + g // 2, (16,))])
                out_ref[b * 32 + k, 16 * g:16 * (g + 1)] = vals


def _tr_body(tt_hbm, tail_hbm, flat_hbm, in_v, out_v, in_tail_v, out_tail_v,
             insem, outsem):
    c = lax.axis_index("c")
    s = lax.axis_index("s")
    wid = s * NC + c
    iota16 = lax.iota(jnp.int32, 16)
    blk0 = wid * BPW

    def in_slice(i):
        return tt_hbm.at[:, pl.ds((blk0 + i * GB) * UNIT, GB * UNIT)]

    def out_slice(i):
        return flat_hbm.at[pl.ds((blk0 + i * GB) * 32, GB * 32)]

    for q in range(NBUF):
        pltpu.async_copy(in_slice(q), in_v.at[q], insem)

    @pl.loop(0, NBATCH)
    def _batch(i):
        p = lax.rem(i, NBUF)
        pltpu.make_async_copy(in_slice(i), in_v.at[p], insem).wait()

        @pl.when(i >= 1)
        def _drain_out():
            pltpu.make_async_copy(out_v, out_slice(i - 1), outsem).wait()

        _tr_compute(in_v.at[p], out_v, iota16)
        pltpu.async_copy(out_v, out_slice(i), outsem)

        @pl.when(i + NBUF < NBATCH)
        def _prefetch():
            pltpu.async_copy(in_slice(i + NBUF), in_v.at[p], insem)

    pltpu.make_async_copy(out_v, out_slice(NBATCH - 1), outsem).wait()

    def _one_block(j):
        pltpu.sync_copy(tt_hbm.at[:, pl.ds(j * UNIT, UNIT)], in_tail_v)

        @pl.loop(0, 32)
        def _row(k):
            for g in range(8):
                vals = plsc.load_gather(
                    in_tail_v,
                    [16 * (g % 2) + iota16,
                     jnp.broadcast_to(4 * k + g // 2, (16,))])
                out_tail_v[k, 16 * g:16 * (g + 1)] = vals
        pltpu.sync_copy(out_tail_v, flat_hbm.at[pl.ds(j * 32, 32)])

    for t in range(LPW):
        _one_block(blk0 + NBATCH * GB + t)

    @pl.when(wid < REM)
    def _rem():
        _one_block(NW * BPW + wid)

    @pl.when(wid == REM)
    def _tail():
        pltpu.sync_copy(tail_hbm, out_tail_v.at[pl.ds(0, 16)])
        pltpu.sync_copy(out_tail_v.at[pl.ds(0, 16)],
                        flat_hbm.at[pl.ds(NBLK * 32, 16)])


def _sc_packed_table(table_t, tail16):
    mesh = plsc.VectorSubcoreMesh(core_axis_name="c", subcore_axis_name="s",
                                  num_cores=NC, num_subcores=NS)
    return pl.kernel(
        _tr_body,
        out_type=jax.ShapeDtypeStruct((V * D // 128, 128), jnp.float32),
        mesh=mesh,
        scratch_types=[
            pltpu.VMEM((NBUF, D, GB * UNIT), jnp.float32),
            pltpu.VMEM((GB * 32, UNIT), jnp.float32),
            pltpu.VMEM((D, UNIT), jnp.float32),
            pltpu.VMEM((D, UNIT), jnp.float32),
            pltpu.SemaphoreType.DMA,
            pltpu.SemaphoreType.DMA,
        ],
        compiler_params=pltpu.CompilerParams(needs_layout_passes=False),
    )(table_t, tail16)


def _sc_body(x_hbm, table_hbm, out_hbm,
             idx_v, dest_v, rows_v, pooled_v, accum_sh, gsem, ssem):
    c = lax.axis_index("c")
    s = lax.axis_index("s")
    wid = s * NC + c

    z = jnp.zeros((16,), jnp.float32)
    for r in range(RPW):
        rows_v[r, 0:16] = z
        rows_v[r, 16:32] = z
    pltpu.sync_copy(rows_v.at[pl.ds(0, RPW)], accum_sh.at[pl.ds(s * RPW, RPW)])

    base0 = wid * IPW
    lane = lax.iota(jnp.int32, 16)
    srow = s * RPW

    @pl.loop(0, CHUNKS)
    def _chunk(i):
        flat0 = base0 + i * CHUNK_ROWS
        pltpu.sync_copy(x_hbm.at[pl.ds(flat0, CHUNK_ROWS)], idx_v)
        for u in range(UPC):
            for k in range(UNIT // 16):
                f = i * CHUNK_ROWS + u * UNIT + k * 16
                dest_v[u, k * 16:(k + 1) * 16] = (
                    srow + lax.div(f + lane, SEQ))
        gathers = [
            pltpu.async_copy(table_hbm.at[idx_v.at[pl.ds(u * UNIT, UNIT)]],
                             rows_v.at[pl.ds(u * UNIT, UNIT)], gsem)
            for u in range(UPC)
        ]
        for g in gathers:
            g.wait()
        scatters = [
            pltpu.async_copy(rows_v.at[pl.ds(u * UNIT, UNIT)],
                             accum_sh.at[dest_v.at[u]], ssem, add=True)
            for u in range(UPC)
        ]
        for t in scatters:
            t.wait()

    pltpu.sync_copy(accum_sh.at[pl.ds(s * RPW, RPW)], pooled_v)
    pltpu.sync_copy(pooled_v, out_hbm.at[pl.ds(wid * RPW, RPW)])


def _sc_pooled_sums(x1, table):
    mesh = plsc.VectorSubcoreMesh(core_axis_name="c", subcore_axis_name="s",
                                  num_cores=NC, num_subcores=NS)
    return pl.kernel(
        _sc_body,
        out_type=jax.ShapeDtypeStruct((B, D), jnp.float32),
        mesh=mesh,
        scratch_types=[
            pltpu.VMEM((CHUNK_ROWS,), jnp.int32),
            pltpu.VMEM((UPC, UNIT), jnp.int32),
            pltpu.VMEM((CHUNK_ROWS, D), jnp.float32),
            pltpu.VMEM((RPW, D), jnp.float32),
            pltpu.VMEM_SHARED((NS * RPW, D), jnp.float32),
            pltpu.SemaphoreType.DMA,
            pltpu.SemaphoreType.DMA,
        ],
        compiler_params=pltpu.CompilerParams(use_tc_tiling_on_sc=False),
    )(x1, table)


def _mm_body(p_ref, w_ref, b_ref, o_ref):
    p = p_ref[...] * (1.0 / SEQ)
    o_ref[...] = lax.dot_general(
        p, w_ref[...], (((1,), (1,)), ((), ())),
        preferred_element_type=jnp.float32) + b_ref[...]


def _classifier(pooled_sums, W, b):
    return pl.pallas_call(
        _mm_body,
        out_shape=jax.ShapeDtypeStruct((B, OUT_DIM), jnp.float32),
    )(pooled_sums, W, b.reshape(1, OUT_DIM))


def kernel(x, emb_table, W, b):
    x1 = x.astype(jnp.int32).reshape(B * SEQ)
    tail16 = emb_table[NBLK * UNIT:].reshape(16, 128)
    packed = _sc_packed_table(emb_table.T, tail16)
    table_lin = packed.reshape(V, D)
    pooled_sums = _sc_pooled_sums(x1, table_lin)
    return _classifier(pooled_sums, W, b)


# speedup vs baseline: 1.5215x; 1.5215x over previous
import jax
import jax.numpy as jnp
from jax import lax
from jax.experimental import pallas as pl
from jax.experimental.pallas import tpu as pltpu
from jax.experimental.pallas import tpu_sc as plsc

B = 4096
SEQ = 200
D = 32
OUT_DIM = 128

NC = 2
NS = 16
NW = NC * NS
RPW = B // NW
IPW = RPW * SEQ
UNIT = 128
UPC = 10
CHUNK_ROWS = UPC * UNIT
CHUNKS = IPW // CHUNK_ROWS


def _sc_body(x_hbm, table_hbm, out_hbm,
             idx_a, idx_b, dest_a, dest_b, rows_a, rows_b, pooled_v,
             accum_sh, gs_a, gs_b, ss_a, ss_b):
    c = lax.axis_index("c")
    s = lax.axis_index("s")
    wid = s * NC + c

    z = jnp.zeros((16,), jnp.float32)
    for r in range(RPW):
        rows_a[r, 0:16] = z
        rows_a[r, 16:32] = z
    pltpu.sync_copy(rows_a.at[pl.ds(0, RPW)], accum_sh.at[pl.ds(s * RPW, RPW)])

    base0 = wid * IPW
    lane = lax.iota(jnp.int32, 16)
    srow = s * RPW

    bufs = ((idx_a, dest_a, rows_a, gs_a, ss_a),
            (idx_b, dest_b, rows_b, gs_b, ss_b))

    def stage(i, bf):
        idx_v, dest_v = bf[0], bf[1]
        pltpu.sync_copy(x_hbm.at[pl.ds(base0 + i * CHUNK_ROWS, CHUNK_ROWS)],
                        idx_v)
        for u in range(UPC):
            for k in range(UNIT // 16):
                f = i * CHUNK_ROWS + u * UNIT + k * 16
                dest_v[u, k * 16:(k + 1) * 16] = srow + lax.div(f + lane, SEQ)

    def fire_g(bf):
        idx_v, rows_v, gsem = bf[0], bf[2], bf[3]
        for u in range(UPC):
            pltpu.async_copy(table_hbm.at[idx_v.at[pl.ds(u * UNIT, UNIT)]],
                             rows_v.at[pl.ds(u * UNIT, UNIT)], gsem)

    def wait_g(bf):
        idx_v, rows_v, gsem = bf[0], bf[2], bf[3]
        for u in range(UPC):
            pltpu.make_async_copy(
                table_hbm.at[idx_v.at[pl.ds(u * UNIT, UNIT)]],
                rows_v.at[pl.ds(u * UNIT, UNIT)], gsem).wait()

    def fire_s(bf):
        dest_v, rows_v, ssem = bf[1], bf[2], bf[4]
        for u in range(UPC):
            pltpu.async_copy(rows_v.at[pl.ds(u * UNIT, UNIT)],
                             accum_sh.at[dest_v.at[u]], ssem, add=True)

    def wait_s(bf):
        dest_v, rows_v, ssem = bf[1], bf[2], bf[4]
        for u in range(UPC):
            pltpu.make_async_copy(rows_v.at[pl.ds(u * UNIT, UNIT)],
                                  accum_sh.at[dest_v.at[u]], ssem).wait()

    stage(0, bufs[0])
    fire_g(bufs[0])
    stage(1, bufs[1])
    fire_g(bufs[1])
    for i in range(CHUNKS):
        bf = bufs[i % 2]
        wait_g(bf)
        fire_s(bf)
        wait_s(bf)
        if i + 2 < CHUNKS:
            stage(i + 2, bf)
            fire_g(bf)

    pltpu.sync_copy(accum_sh.at[pl.ds(s * RPW, RPW)], pooled_v)
    pltpu.sync_copy(pooled_v, out_hbm.at[pl.ds(wid * RPW, RPW)])


def _sc_pooled_sums(x1, table):
    mesh = plsc.VectorSubcoreMesh(core_axis_name="c", subcore_axis_name="s",
                                  num_cores=NC, num_subcores=NS)
    return pl.kernel(
        _sc_body,
        out_type=jax.ShapeDtypeStruct((B, D), jnp.float32),
        mesh=mesh,
        scratch_types=[
            pltpu.VMEM((CHUNK_ROWS,), jnp.int32),
            pltpu.VMEM((CHUNK_ROWS,), jnp.int32),
            pltpu.VMEM((UPC, UNIT), jnp.int32),
            pltpu.VMEM((UPC, UNIT), jnp.int32),
            pltpu.VMEM((CHUNK_ROWS, D), jnp.float32),
            pltpu.VMEM((CHUNK_ROWS, D), jnp.float32),
            pltpu.VMEM((RPW, D), jnp.float32),
            pltpu.VMEM_SHARED((NS * RPW, D), jnp.float32),
            pltpu.SemaphoreType.DMA,
            pltpu.SemaphoreType.DMA,
            pltpu.SemaphoreType.DMA,
            pltpu.SemaphoreType.DMA,
        ],
        compiler_params=pltpu.CompilerParams(use_tc_tiling_on_sc=False),
    )(x1, table)


def _mm_body(p_ref, w_ref, b_ref, o_ref):
    p = p_ref[...] * (1.0 / SEQ)
    o_ref[...] = lax.dot_general(
        p, w_ref[...], (((1,), (1,)), ((), ())),
        preferred_element_type=jnp.float32) + b_ref[...]


def _classifier(pooled_sums, W, b):
    return pl.pallas_call(
        _mm_body,
        out_shape=jax.ShapeDtypeStruct((B, OUT_DIM), jnp.float32),
    )(pooled_sums, W, b.reshape(1, OUT_DIM))


def kernel(x, emb_table, W, b):
    x1 = x.astype(jnp.int32).reshape(B * SEQ)
    pooled_sums = _sc_pooled_sums(x1, emb_table)
    return _classifier(pooled_sums, W, b)
